# baseline (speedup 1.0000x reference)
import jax
import jax.numpy as jnp
import numpy as np
from jax.experimental import pallas as pl

_COST_CLASS = 1.0
_COST_BBOX = 5.0
_COST_GIOU = 2.0


def _cost_block_kernel(lg_ref, bx_ref, ids_ref, tbx_ref, out_ref):
    lg = lg_ref[0]
    nc = lg.shape[-1]
    nt = ids_ref.shape[-1]

    m = jnp.max(lg, axis=-1, keepdims=True)
    e = jnp.exp(lg - m)
    p = e * (1.0 / jnp.sum(e, axis=-1, keepdims=True))

    ids = ids_ref[...]
    cls_iota = jax.lax.broadcasted_iota(jnp.int32, (nc, nt), 0)
    M = jnp.where(
        cls_iota == jnp.broadcast_to(ids, (nc, nt)), -_COST_CLASS, 0.0,
    )
    pM = jax.lax.dot_general(
        p, M, (((1,), (0,)), ((), ())),
        precision=jax.lax.Precision.HIGHEST,
        preferred_element_type=jnp.float32,
    )

    bxq = bx_ref[0]
    cx = bxq[:, 0:1]
    cy = bxq[:, 1:2]
    w = bxq[:, 2:3]
    h = bxq[:, 3:4]
    tcx = tbx_ref[0:1, :]
    tcy = tbx_ref[1:2, :]
    tw = tbx_ref[2:3, :]
    th = tbx_ref[3:4, :]

    l1 = (
        jnp.abs(cx - tcx) + jnp.abs(cy - tcy)
        + jnp.abs(w - tw) + jnp.abs(h - th)
    )

    x0 = cx - 0.5 * w
    y0 = cy - 0.5 * h
    x1 = cx + 0.5 * w
    y1 = cy + 0.5 * h
    tx0 = tcx - 0.5 * tw
    ty0 = tcy - 0.5 * th
    tx1 = tcx + 0.5 * tw
    ty1 = tcy + 0.5 * th

    sw = jnp.minimum(x1, tx1) - jnp.maximum(x0, tx0)
    sh = jnp.minimum(y1, ty1) - jnp.maximum(y0, ty0)
    inter = jnp.maximum(sw, 0.0) * jnp.maximum(sh, 0.0)
    enc = ((w + tw) - sw) * ((h + th) - sh)
    area1 = (x1 - x0) * (y1 - y0)
    area2 = (tx1 - tx0) * (ty1 - ty0)
    union = (area1 + area2) - inter
    ab = inter / union + union / enc
    out_ref[0] = (_COST_BBOX * l1 + pM) - _COST_GIOU * (ab - 1.0)


def _lsa_np(cost):
    cost = np.asarray(cost, dtype=np.float64)
    transposed = False
    if cost.shape[0] > cost.shape[1]:
        cost = cost.T
        transposed = True
    n, m = cost.shape
    INF = 1e18
    u = np.zeros(n + 1)
    v = np.zeros(m + 1)
    p = np.zeros(m + 1, dtype=np.int64)
    way = np.zeros(m + 1, dtype=np.int64)
    for i in range(1, n + 1):
        p[0] = i
        j0 = 0
        minv = np.full(m + 1, INF)
        used = np.zeros(m + 1, dtype=bool)
        while True:
            used[j0] = True
            i0 = p[j0]
            cur = cost[i0 - 1, :] - u[i0] - v[1:]
            free = ~used[1:]
            better = free & (cur < minv[1:])
            idx = np.nonzero(better)[0] + 1
            minv[idx] = cur[idx - 1]
            way[idx] = j0
            cand = np.where(free, minv[1:], INF)
            j1 = int(np.argmin(cand)) + 1
            delta = cand[j1 - 1]
            u[p[used]] += delta
            v[used] -= delta
            freeidx = np.nonzero(free)[0] + 1
            minv[freeidx] -= delta
            j0 = j1
            if p[j0] == 0:
                break
        while j0 != 0:
            j1 = int(way[j0])
            p[j0] = p[j1]
            j0 = j1
    col4row = np.full(n, -1, dtype=np.int64)
    for j in range(1, m + 1):
        if p[j] != 0:
            col4row[p[j] - 1] = j - 1
    row_ind = np.arange(n, dtype=np.int64)
    col_ind = col4row
    if transposed:
        row_ind, col_ind = col_ind, row_ind
        order = np.argsort(row_ind)
        row_ind = row_ind[order]
        col_ind = col_ind[order]
    return row_ind, col_ind


def _assign_batched(Cn):
    Cn = np.asarray(Cn)
    bs, nq, total = Cn.shape
    nt = total // bs
    rows, cols = [], []
    for b in range(bs):
        r, c = _lsa_np(Cn[b, :, b * nt:(b + 1) * nt])
        rows.append(r)
        cols.append(c)
    return np.stack(rows).astype(np.int32), np.stack(cols).astype(np.int32)


def _cost_matrix_pallas(pred_logits, pred_boxes, tgt_labels, tgt_boxes):
    bs, nq, nc = pred_logits.shape
    nt = tgt_labels.shape[0] * tgt_labels.shape[1]
    ids = tgt_labels.reshape(1, nt)
    tbx = tgt_boxes.reshape(nt, 4).T

    bq = 1000
    while nq % bq or bq % 8:
        bq //= 2

    return pl.pallas_call(
        _cost_block_kernel,
        grid=(bs, nq // bq),
        in_specs=[
            pl.BlockSpec((1, bq, nc), lambda b, i: (b, i, 0)),
            pl.BlockSpec((1, bq, 4), lambda b, i: (b, i, 0)),
            pl.BlockSpec((1, nt), lambda b, i: (0, 0)),
            pl.BlockSpec((4, nt), lambda b, i: (0, 0)),
        ],
        out_specs=pl.BlockSpec((1, bq, nt), lambda b, i: (b, i, 0)),
        out_shape=jax.ShapeDtypeStruct((bs, nq, nt), jnp.float32),
    )(pred_logits, pred_boxes, ids, tbx)


def kernel(pred_logits, pred_boxes, tgt_labels, tgt_boxes):
    bs, nq, _ = pred_logits.shape
    C = _cost_matrix_pallas(pred_logits, pred_boxes, tgt_labels, tgt_boxes)
    k = min(nq, tgt_labels.shape[1])
    result_shapes = (
        jax.ShapeDtypeStruct((bs, k), jnp.int32),
        jax.ShapeDtypeStruct((bs, k), jnp.int32),
    )
    ind_i, ind_j = jax.pure_callback(_assign_batched, result_shapes, C)
    return (ind_i, ind_j, C)


# speedup vs baseline: 1.0459x; 1.0459x over previous
import jax
import jax.numpy as jnp
import numpy as np
from jax.experimental import pallas as pl

_COST_CLASS = 1.0
_COST_BBOX = 5.0
_COST_GIOU = 2.0


def _cost_block_kernel(lg_ref, bx_ref, ids_ref, tbx_ref, out_ref):
    lg = lg_ref[0]
    nc = lg.shape[-1]
    nt = ids_ref.shape[-1]

    m = jnp.max(lg, axis=-1, keepdims=True)
    e = jnp.exp(lg - m)
    p = e * (1.0 / jnp.sum(e, axis=-1, keepdims=True))

    ids = ids_ref[...]
    cls_iota = jax.lax.broadcasted_iota(jnp.int32, (nc, nt), 0)
    M = jnp.where(
        cls_iota == jnp.broadcast_to(ids, (nc, nt)), -_COST_CLASS, 0.0,
    ).astype(jnp.bfloat16)
    p_hi = p.astype(jnp.bfloat16)
    p_lo = (p - p_hi.astype(jnp.float32)).astype(jnp.bfloat16)
    dn = (((1,), (0,)), ((), ()))
    pM = (
        jax.lax.dot_general(p_hi, M, dn, preferred_element_type=jnp.float32)
        + jax.lax.dot_general(p_lo, M, dn, preferred_element_type=jnp.float32)
    )

    bxq = bx_ref[0]
    cx = bxq[:, 0:1]
    cy = bxq[:, 1:2]
    w = bxq[:, 2:3]
    h = bxq[:, 3:4]
    tcx = tbx_ref[0:1, :]
    tcy = tbx_ref[1:2, :]
    tw = tbx_ref[2:3, :]
    th = tbx_ref[3:4, :]

    l1 = (
        jnp.abs(cx - tcx) + jnp.abs(cy - tcy)
        + jnp.abs(w - tw) + jnp.abs(h - th)
    )

    x0 = cx - 0.5 * w
    y0 = cy - 0.5 * h
    x1 = cx + 0.5 * w
    y1 = cy + 0.5 * h
    tx0 = tcx - 0.5 * tw
    ty0 = tcy - 0.5 * th
    tx1 = tcx + 0.5 * tw
    ty1 = tcy + 0.5 * th

    sw = jnp.minimum(x1, tx1) - jnp.maximum(x0, tx0)
    sh = jnp.minimum(y1, ty1) - jnp.maximum(y0, ty0)
    inter = jnp.maximum(sw, 0.0) * jnp.maximum(sh, 0.0)
    enc = ((w + tw) - sw) * ((h + th) - sh)
    area1 = (x1 - x0) * (y1 - y0)
    area2 = (tx1 - tx0) * (ty1 - ty0)
    union = (area1 + area2) - inter
    ab = inter / union + union / enc
    out_ref[0] = (_COST_BBOX * l1 + pM) - _COST_GIOU * (ab - 1.0)


def _lsa_np(cost):
    cost = np.asarray(cost, dtype=np.float64)
    transposed = False
    if cost.shape[0] > cost.shape[1]:
        cost = cost.T
        transposed = True
    n, m = cost.shape
    INF = 1e18
    u = np.zeros(n + 1)
    v = np.zeros(m + 1)
    p = np.zeros(m + 1, dtype=np.int64)
    way = np.zeros(m + 1, dtype=np.int64)
    for i in range(1, n + 1):
        p[0] = i
        j0 = 0
        minv = np.full(m + 1, INF)
        used = np.zeros(m + 1, dtype=bool)
        while True:
            used[j0] = True
            i0 = p[j0]
            cur = cost[i0 - 1, :] - u[i0] - v[1:]
            free = ~used[1:]
            better = free & (cur < minv[1:])
            idx = np.nonzero(better)[0] + 1
            minv[idx] = cur[idx - 1]
            way[idx] = j0
            cand = np.where(free, minv[1:], INF)
            j1 = int(np.argmin(cand)) + 1
            delta = cand[j1 - 1]
            u[p[used]] += delta
            v[used] -= delta
            freeidx = np.nonzero(free)[0] + 1
            minv[freeidx] -= delta
            j0 = j1
            if p[j0] == 0:
                break
        while j0 != 0:
            j1 = int(way[j0])
            p[j0] = p[j1]
            j0 = j1
    col4row = np.full(n, -1, dtype=np.int64)
    for j in range(1, m + 1):
        if p[j] != 0:
            col4row[p[j] - 1] = j - 1
    row_ind = np.arange(n, dtype=np.int64)
    col_ind = col4row
    if transposed:
        row_ind, col_ind = col_ind, row_ind
        order = np.argsort(row_ind)
        row_ind = row_ind[order]
        col_ind = col_ind[order]
    return row_ind, col_ind


def _assign_batched(Cn):
    Cn = np.asarray(Cn)
    bs, nq, total = Cn.shape
    nt = total // bs
    rows, cols = [], []
    for b in range(bs):
        r, c = _lsa_np(Cn[b, :, b * nt:(b + 1) * nt])
        rows.append(r)
        cols.append(c)
    return np.stack(rows).astype(np.int32), np.stack(cols).astype(np.int32)


def _cost_matrix_pallas(pred_logits, pred_boxes, tgt_labels, tgt_boxes):
    bs, nq, nc = pred_logits.shape
    nt = tgt_labels.shape[0] * tgt_labels.shape[1]
    ids = tgt_labels.reshape(1, nt)
    tbx = tgt_boxes.reshape(nt, 4).T

    bq = 1000
    while nq % bq or bq % 8:
        bq //= 2

    return pl.pallas_call(
        _cost_block_kernel,
        grid=(bs, nq // bq),
        in_specs=[
            pl.BlockSpec((1, bq, nc), lambda b, i: (b, i, 0)),
            pl.BlockSpec((1, bq, 4), lambda b, i: (b, i, 0)),
            pl.BlockSpec((1, nt), lambda b, i: (0, 0)),
            pl.BlockSpec((4, nt), lambda b, i: (0, 0)),
        ],
        out_specs=pl.BlockSpec((1, bq, nt), lambda b, i: (b, i, 0)),
        out_shape=jax.ShapeDtypeStruct((bs, nq, nt), jnp.float32),
    )(pred_logits, pred_boxes, ids, tbx)


def kernel(pred_logits, pred_boxes, tgt_labels, tgt_boxes):
    bs, nq, _ = pred_logits.shape
    C = _cost_matrix_pallas(pred_logits, pred_boxes, tgt_labels, tgt_boxes)
    k = min(nq, tgt_labels.shape[1])
    result_shapes = (
        jax.ShapeDtypeStruct((bs, k), jnp.int32),
        jax.ShapeDtypeStruct((bs, k), jnp.int32),
    )
    ind_i, ind_j = jax.pure_callback(_assign_batched, result_shapes, C)
    return (ind_i, ind_j, C)
